# Initial kernel scaffold; baseline (speedup 1.0000x reference)
import functools

import jax
import jax.numpy as jnp
from jax import lax
from jax.experimental import pallas as pl
from jax.experimental.pallas import tpu as pltpu
from jax.experimental.pallas import tpu_sc as plsc

_NC = 2
_NS = 16
_NW = _NC * _NS
_CH = 128
_B = 64


def _pad_rows(n):
    return ((n + _NW - 1) // _NW + 7) // 8 * 8 * _NS


@functools.cache
def _make_deg_kernel(n_acc, n_chunks):
    mesh = plsc.VectorSubcoreMesh(core_axis_name="c", subcore_axis_name="s")
    chunks_per_tile = n_chunks // _NW
    rows_per_tile = n_acc // _NS

    @functools.partial(
        pl.kernel,
        mesh=mesh,
        out_type=jax.ShapeDtypeStruct((_NC, n_acc), jnp.float32),
        scratch_types=[
            pltpu.VMEM((_CH,), jnp.int32),
            pltpu.VMEM((_CH,), jnp.float32),
            pltpu.MemoryRef((n_acc,), jnp.float32, memory_space=pltpu.VMEM_SHARED),
        ],
    )
    def deg_kernel(zeros_hbm, dst_hbm, out_hbm, dst_v, ones_v, acc_sh):
        cid = lax.axis_index("c")
        sid = lax.axis_index("s")
        base_row = sid * rows_per_tile
        pltpu.sync_copy(zeros_hbm.at[pl.ds(base_row, rows_per_tile)],
                        acc_sh.at[pl.ds(base_row, rows_per_tile)])
        for i in range(_CH // 16):
            ones_v[pl.ds(i * 16, 16)] = jnp.ones((16,), jnp.float32)
        plsc.subcore_barrier()
        wid = sid * _NC + cid
        chunk0 = wid * chunks_per_tile

        def body(j, carry):
            pltpu.sync_copy(dst_hbm.at[chunk0 + j], dst_v)
            pltpu.sync_copy(ones_v, acc_sh.at[dst_v], add=True)
            return carry

        lax.fori_loop(0, chunks_per_tile, body, 0)
        plsc.subcore_barrier()
        pltpu.sync_copy(acc_sh.at[pl.ds(base_row, rows_per_tile)],
                        out_hbm.at[cid, pl.ds(base_row, rows_per_tile)])

    return deg_kernel


@functools.cache
def _make_edge_kernel(n_acc, d_feat, n_chunks):
    mesh = plsc.VectorSubcoreMesh(core_axis_name="c", subcore_axis_name="s")
    chunks_per_tile = n_chunks // _NW
    rows_per_tile = n_acc // _NS

    @functools.partial(
        pl.kernel,
        mesh=mesh,
        out_type=jax.ShapeDtypeStruct((_NC, n_acc, d_feat), jnp.float32),
        scratch_types=[
            pltpu.VMEM((_CH,), jnp.int32),
            pltpu.VMEM((_CH,), jnp.int32),
            pltpu.VMEM((_CH, d_feat), jnp.float32),
            pltpu.MemoryRef((n_acc, d_feat), jnp.float32,
                            memory_space=pltpu.VMEM_SHARED),
            pltpu.SemaphoreType.DMA,
        ],
    )
    def edge_kernel(g_hbm, src_hbm, dst_hbm, out_hbm,
                    src_v, dst_v, rows_v, acc_sh, sem):
        cid = lax.axis_index("c")
        sid = lax.axis_index("s")
        base_row = sid * rows_per_tile
        pltpu.sync_copy(g_hbm.at[pl.ds(base_row, rows_per_tile)],
                        acc_sh.at[pl.ds(base_row, rows_per_tile)])
        plsc.subcore_barrier()
        wid = sid * _NC + cid
        chunk0 = wid * chunks_per_tile

        def body(j, carry):
            pltpu.sync_copy(src_hbm.at[chunk0 + j], src_v)
            pltpu.async_copy(g_hbm.at[src_v], rows_v, sem).wait()
            pltpu.sync_copy(dst_hbm.at[chunk0 + j], dst_v)
            pltpu.sync_copy(rows_v, acc_sh.at[dst_v], add=True)
            return carry

        lax.fori_loop(0, chunks_per_tile, body, 0)
        plsc.subcore_barrier()
        pltpu.sync_copy(acc_sh.at[pl.ds(base_row, rows_per_tile)],
                        out_hbm.at[cid, pl.ds(base_row, rows_per_tile)])

    return edge_kernel


def _k1_body(degp_ref, x_ref, w1_ref, g1_ref, d_ref):
    deg = degp_ref[0] + degp_ref[1] + 1.0
    d = lax.rsqrt(deg)
    h = jnp.dot(x_ref[:], w1_ref[:], preferred_element_type=jnp.float32)
    g1_ref[:] = h * d
    d_ref[:] = d


def _k2_body(p_ref, g1_ref, d_ref, b1_ref, w2_ref, g2_ref):
    tot = p_ref[0] + p_ref[1] - g1_ref[:]
    a1 = jnp.maximum(d_ref[:] * tot + b1_ref[:], 0.0)
    h2 = jnp.dot(a1, w2_ref[:], preferred_element_type=jnp.float32)
    g2_ref[:] = h2 * d_ref[:]


def _k3_body(n_acc, p_ref, g2_ref, d_ref, b2_ref, batch_ref, wout_ref,
             bout_ref, out_ref):
    tot = p_ref[0] + p_ref[1] - g2_ref[:]
    a2 = jnp.maximum(d_ref[:] * tot + b2_ref[:], 0.0)
    ids = lax.broadcasted_iota(jnp.int32, (_B, n_acc), 0)
    oneh = (ids == batch_ref[:]).astype(jnp.float32)
    sums = jnp.dot(oneh, a2, preferred_element_type=jnp.float32)
    counts = jnp.sum(oneh, axis=1, keepdims=True)
    pooled = sums / jnp.maximum(counts, 1.0)
    out_ref[:] = (jnp.dot(pooled, wout_ref[:],
                          preferred_element_type=jnp.float32) + bout_ref[:])


def kernel(x, edge_index, batch, W1, b1, W2, b2, Wout, bout):
    n, f_in = x.shape
    e = edge_index.shape[1]
    n_acc = _pad_rows(n)
    e_pad = (e + _NW * _CH - 1) // (_NW * _CH) * (_NW * _CH)
    n_chunks = e_pad // _CH

    f32 = jnp.float32
    x_pad = jnp.zeros((n_acc, f_in), f32).at[:n].set(x)
    pad_idx = jnp.full((e_pad - e,), n, jnp.int32)
    src_chunks = jnp.concatenate([edge_index[0], pad_idx]).reshape(n_chunks, _CH)
    dst_chunks = jnp.concatenate([edge_index[1], pad_idx]).reshape(n_chunks, _CH)
    batch_pad = jnp.full((1, n_acc), _B, jnp.int32).at[0, :n].set(batch)
    zeros_deg = jnp.zeros((n_acc,), f32)

    deg_p = _make_deg_kernel(n_acc, n_chunks)(zeros_deg, dst_chunks)
    deg_p = deg_p.reshape(_NC, n_acc, 1)

    d_out = W1.shape[1]
    g1, d = pl.pallas_call(
        _k1_body,
        out_shape=[jax.ShapeDtypeStruct((n_acc, d_out), f32),
                   jax.ShapeDtypeStruct((n_acc, 1), f32)],
    )(deg_p, x_pad, W1)

    p1 = _make_edge_kernel(n_acc, d_out, n_chunks)(g1, src_chunks, dst_chunks)

    d2_out = W2.shape[1]
    g2 = pl.pallas_call(
        _k2_body,
        out_shape=jax.ShapeDtypeStruct((n_acc, d2_out), f32),
    )(p1, g1, d, b1.reshape(1, -1), W2)

    p2 = _make_edge_kernel(n_acc, d2_out, n_chunks)(g2, src_chunks, dst_chunks)

    out = pl.pallas_call(
        functools.partial(_k3_body, n_acc),
        out_shape=jax.ShapeDtypeStruct((_B, Wout.shape[1]), f32),
    )(p2, g2, d, b2.reshape(1, -1), batch_pad, Wout, bout.reshape(1, -1))
    return out


# speedup vs baseline: 17.0990x; 17.0990x over previous
import functools

import jax
import jax.numpy as jnp
from jax import lax
from jax.experimental import pallas as pl
from jax.experimental.pallas import tpu as pltpu
from jax.experimental.pallas import tpu_sc as plsc

_NC = 2
_NS = 16
_NW = _NC * _NS
_CH = 128
_B = 64


def _pad_rows(n):
    return (n + 127) // 128 * 128


@functools.cache
def _make_deg_kernel(n_acc, n_chunks):
    mesh = plsc.VectorSubcoreMesh(core_axis_name="c", subcore_axis_name="s",
                                  num_cores=_NC, num_subcores=_NS)
    chunks_per_tile = n_chunks // _NW
    rows_per_tile = n_acc // _NS

    @functools.partial(
        pl.kernel,
        mesh=mesh,
        out_type=jax.ShapeDtypeStruct((_NC * n_acc,), jnp.float32),
        scratch_types=[
            pltpu.VMEM((_CH,), jnp.int32),
            pltpu.VMEM((_CH,), jnp.float32),
            pltpu.VMEM((n_acc // _NS,), jnp.float32),
            pltpu.VMEM_SHARED((n_acc,), jnp.float32),
        ],
    )
    def deg_kernel(zeros_hbm, dst_hbm, out_hbm, dst_v, ones_v, tmp_v, acc_sh):
        cid = lax.axis_index("c")
        sid = lax.axis_index("s")
        base_row = sid * rows_per_tile
        pltpu.sync_copy(zeros_hbm.at[pl.ds(base_row, rows_per_tile)], tmp_v)
        pltpu.sync_copy(tmp_v, acc_sh.at[pl.ds(base_row, rows_per_tile)])
        for i in range(_CH // 16):
            ones_v[pl.ds(i * 16, 16)] = jnp.ones((16,), jnp.float32)
        plsc.subcore_barrier()
        wid = sid * _NC + cid
        chunk0 = wid * chunks_per_tile

        def body(j, carry):
            pltpu.sync_copy(dst_hbm.at[chunk0 + j], dst_v)
            pltpu.sync_copy(ones_v, acc_sh.at[dst_v], add=True)
            return carry

        lax.fori_loop(0, chunks_per_tile, body, 0)
        plsc.subcore_barrier()
        pltpu.sync_copy(acc_sh.at[pl.ds(base_row, rows_per_tile)], tmp_v)
        pltpu.sync_copy(tmp_v,
                        out_hbm.at[pl.ds(cid * n_acc + base_row, rows_per_tile)])

    return deg_kernel


@functools.cache
def _make_edge_kernel(n_acc, d_feat, n_chunks):
    mesh = plsc.VectorSubcoreMesh(core_axis_name="c", subcore_axis_name="s",
                                  num_cores=_NC, num_subcores=_NS)
    chunks_per_tile = n_chunks // _NW
    rows_per_tile = n_acc // _NS

    @functools.partial(
        pl.kernel,
        mesh=mesh,
        out_type=jax.ShapeDtypeStruct((_NC, n_acc, d_feat), jnp.float32),
        scratch_types=[
            pltpu.VMEM((_CH,), jnp.int32),
            pltpu.VMEM((_CH,), jnp.int32),
            pltpu.VMEM((_CH, d_feat), jnp.float32),
            pltpu.VMEM_SHARED((n_acc, d_feat), jnp.float32),
            pltpu.SemaphoreType.DMA,
        ],
        compiler_params=pltpu.CompilerParams(use_tc_tiling_on_sc=False),
    )
    def edge_kernel(g_hbm, src_hbm, dst_hbm, out_hbm,
                    src_v, dst_v, rows_v, acc_sh, sem):
        cid = lax.axis_index("c")
        sid = lax.axis_index("s")
        base_row = sid * rows_per_tile
        pltpu.sync_copy(g_hbm.at[pl.ds(base_row, rows_per_tile)],
                        acc_sh.at[pl.ds(base_row, rows_per_tile)])
        plsc.subcore_barrier()
        wid = sid * _NC + cid
        chunk0 = wid * chunks_per_tile

        def body(j, carry):
            pltpu.sync_copy(src_hbm.at[chunk0 + j], src_v)
            pltpu.async_copy(g_hbm.at[src_v], rows_v, sem).wait()
            pltpu.sync_copy(dst_hbm.at[chunk0 + j], dst_v)
            pltpu.sync_copy(rows_v, acc_sh.at[dst_v], add=True)
            return carry

        lax.fori_loop(0, chunks_per_tile, body, 0)
        plsc.subcore_barrier()
        pltpu.sync_copy(acc_sh.at[pl.ds(base_row, rows_per_tile)],
                        out_hbm.at[cid, pl.ds(base_row, rows_per_tile)])

    return edge_kernel


def _k1_body(degp_ref, x_ref, w1_ref, g1_ref, d_ref):
    deg = degp_ref[0] + degp_ref[1] + 1.0
    d = lax.rsqrt(deg)
    h = jnp.dot(x_ref[:], w1_ref[:], preferred_element_type=jnp.float32)
    g1_ref[:] = h * d
    d_ref[:] = d


def _k2_body(p_ref, g1_ref, d_ref, b1_ref, w2_ref, g2_ref):
    tot = p_ref[0] + p_ref[1] - g1_ref[:]
    a1 = jnp.maximum(d_ref[:] * tot + b1_ref[:], 0.0)
    h2 = jnp.dot(a1, w2_ref[:], preferred_element_type=jnp.float32)
    g2_ref[:] = h2 * d_ref[:]


def _k3_body(n_acc, p_ref, g2_ref, d_ref, b2_ref, batch_ref, wout_ref,
             bout_ref, out_ref):
    tot = p_ref[0] + p_ref[1] - g2_ref[:]
    a2 = jnp.maximum(d_ref[:] * tot + b2_ref[:], 0.0)
    ids = lax.broadcasted_iota(jnp.int32, (_B, n_acc), 0)
    oneh = (ids == batch_ref[:]).astype(jnp.float32)
    sums = jnp.dot(oneh, a2, preferred_element_type=jnp.float32)
    counts = jnp.sum(oneh, axis=1, keepdims=True)
    pooled = sums / jnp.maximum(counts, 1.0)
    out_ref[:] = (jnp.dot(pooled, wout_ref[:],
                          preferred_element_type=jnp.float32) + bout_ref[:])


def kernel(x, edge_index, batch, W1, b1, W2, b2, Wout, bout):
    n, f_in = x.shape
    e = edge_index.shape[1]
    n_acc = _pad_rows(n)
    e_pad = (e + _NW * _CH - 1) // (_NW * _CH) * (_NW * _CH)
    n_chunks = e_pad // _CH

    f32 = jnp.float32
    x_pad = jnp.zeros((n_acc, f_in), f32).at[:n].set(x)
    pad_idx = jnp.full((e_pad - e,), n, jnp.int32)
    src_chunks = jnp.concatenate([edge_index[0], pad_idx]).reshape(n_chunks, _CH)
    dst_chunks = jnp.concatenate([edge_index[1], pad_idx]).reshape(n_chunks, _CH)
    batch_pad = jnp.full((1, n_acc), _B, jnp.int32).at[0, :n].set(batch)
    zeros_deg = jnp.zeros((n_acc,), f32)

    deg_p = _make_deg_kernel(n_acc, n_chunks)(zeros_deg, dst_chunks)
    deg_p = deg_p.reshape(_NC, n_acc, 1)

    d_out = W1.shape[1]
    g1, d = pl.pallas_call(
        _k1_body,
        out_shape=[jax.ShapeDtypeStruct((n_acc, d_out), f32),
                   jax.ShapeDtypeStruct((n_acc, 1), f32)],
    )(deg_p, x_pad, W1)

    p1 = _make_edge_kernel(n_acc, d_out, n_chunks)(g1, src_chunks, dst_chunks)

    d2_out = W2.shape[1]
    g2 = pl.pallas_call(
        _k2_body,
        out_shape=jax.ShapeDtypeStruct((n_acc, d2_out), f32),
    )(p1, g1, d, b1.reshape(1, -1), W2)

    p2 = _make_edge_kernel(n_acc, d2_out, n_chunks)(g2, src_chunks, dst_chunks)

    out = pl.pallas_call(
        functools.partial(_k3_body, n_acc),
        out_shape=jax.ShapeDtypeStruct((_B, Wout.shape[1]), f32),
    )(p2, g2, d, b2.reshape(1, -1), batch_pad, Wout, bout.reshape(1, -1))
    return out


# speedup vs baseline: 21.5594x; 1.2609x over previous
import functools

import jax
import jax.numpy as jnp
from jax import lax
from jax.experimental import pallas as pl
from jax.experimental.pallas import tpu as pltpu
from jax.experimental.pallas import tpu_sc as plsc

_NC = 2
_NS = 16
_NW = _NC * _NS
_CH = 128
_B = 64


def _pad_rows(n):
    return (n + 127) // 128 * 128


@functools.cache
def _make_deg_kernel(n_acc, n_chunks):
    mesh = plsc.VectorSubcoreMesh(core_axis_name="c", subcore_axis_name="s",
                                  num_cores=_NC, num_subcores=_NS)
    chunks_per_tile = n_chunks // _NW
    rows_per_tile = n_acc // _NS

    @functools.partial(
        pl.kernel,
        mesh=mesh,
        out_type=jax.ShapeDtypeStruct((_NC * n_acc,), jnp.float32),
        scratch_types=[
            pltpu.VMEM((chunks_per_tile, _CH), jnp.int32),
            pltpu.VMEM((_CH,), jnp.float32),
            pltpu.VMEM((n_acc // _NS,), jnp.float32),
            pltpu.VMEM_SHARED((n_acc,), jnp.float32),
            pltpu.SemaphoreType.DMA,
        ],
        compiler_params=pltpu.CompilerParams(use_tc_tiling_on_sc=False),
    )
    def deg_kernel(zeros_hbm, dst_hbm, out_hbm, dst_all, ones_v, tmp_v,
                   acc_sh, sem):
        cid = lax.axis_index("c")
        sid = lax.axis_index("s")
        base_row = sid * rows_per_tile
        wid = sid * _NC + cid
        chunk0 = wid * chunks_per_tile
        pltpu.sync_copy(dst_hbm.at[pl.ds(chunk0, chunks_per_tile)], dst_all)
        pltpu.sync_copy(zeros_hbm.at[pl.ds(base_row, rows_per_tile)], tmp_v)
        pltpu.sync_copy(tmp_v, acc_sh.at[pl.ds(base_row, rows_per_tile)])
        for i in range(_CH // 16):
            ones_v[pl.ds(i * 16, 16)] = jnp.ones((16,), jnp.float32)
        plsc.subcore_barrier()

        def fire(j, carry):
            pltpu.async_copy(ones_v, acc_sh.at[dst_all.at[j]], sem, add=True)
            return carry

        lax.fori_loop(0, chunks_per_tile, fire, 0)

        def drain(j, carry):
            pltpu.make_async_copy(ones_v, acc_sh.at[dst_all.at[0]], sem).wait()
            return carry

        lax.fori_loop(0, chunks_per_tile, drain, 0)
        plsc.subcore_barrier()
        pltpu.sync_copy(acc_sh.at[pl.ds(base_row, rows_per_tile)], tmp_v)
        pltpu.sync_copy(tmp_v,
                        out_hbm.at[pl.ds(cid * n_acc + base_row, rows_per_tile)])

    return deg_kernel


@functools.cache
def _make_edge_kernel(n_acc, d_feat, n_chunks):
    mesh = plsc.VectorSubcoreMesh(core_axis_name="c", subcore_axis_name="s",
                                  num_cores=_NC, num_subcores=_NS)
    chunks_per_tile = n_chunks // _NW
    rows_per_tile = n_acc // _NS

    @functools.partial(
        pl.kernel,
        mesh=mesh,
        out_type=jax.ShapeDtypeStruct((_NC, n_acc, d_feat), jnp.float32),
        scratch_types=[
            pltpu.VMEM((chunks_per_tile, _CH), jnp.int32),
            pltpu.VMEM((chunks_per_tile, _CH), jnp.int32),
            pltpu.VMEM((_CH, d_feat), jnp.float32),
            pltpu.VMEM((_CH, d_feat), jnp.float32),
            pltpu.VMEM_SHARED((n_acc, d_feat), jnp.float32),
            pltpu.SemaphoreType.DMA,
            pltpu.SemaphoreType.DMA,
            pltpu.SemaphoreType.DMA,
            pltpu.SemaphoreType.DMA,
        ],
        compiler_params=pltpu.CompilerParams(use_tc_tiling_on_sc=False),
    )
    def edge_kernel(g_hbm, src_hbm, dst_hbm, out_hbm,
                    src_all, dst_all, rows0, rows1, acc_sh,
                    gsem0, gsem1, ssem0, ssem1):
        cid = lax.axis_index("c")
        sid = lax.axis_index("s")
        base_row = sid * rows_per_tile
        wid = sid * _NC + cid
        chunk0 = wid * chunks_per_tile
        rows = (rows0, rows1)
        gsem = (gsem0, gsem1)
        ssem = (ssem0, ssem1)
        pltpu.sync_copy(src_hbm.at[pl.ds(chunk0, chunks_per_tile)], src_all)
        pltpu.sync_copy(dst_hbm.at[pl.ds(chunk0, chunks_per_tile)], dst_all)
        pltpu.sync_copy(g_hbm.at[pl.ds(base_row, rows_per_tile)],
                        acc_sh.at[pl.ds(base_row, rows_per_tile)])
        plsc.subcore_barrier()

        def gstart(j, b):
            pltpu.async_copy(g_hbm.at[src_all.at[j]], rows[b], gsem[b])

        def gwait(j, b):
            pltpu.make_async_copy(g_hbm.at[src_all.at[j]], rows[b],
                                  gsem[b]).wait()

        def sstart(j, b):
            pltpu.async_copy(rows[b], acc_sh.at[dst_all.at[j]], ssem[b],
                             add=True)

        def swait(j, b):
            pltpu.make_async_copy(rows[b], acc_sh.at[dst_all.at[j]],
                                  ssem[b]).wait()

        gstart(0, 0)
        gstart(1, 1)

        def body(k, carry):
            j0 = 2 * k
            j1 = 2 * k + 1
            gwait(j0, 0)
            sstart(j0, 0)
            gwait(j1, 1)
            sstart(j1, 1)
            swait(j0, 0)
            gstart(j0 + 2, 0)
            swait(j1, 1)
            gstart(j1 + 2, 1)
            return carry

        lax.fori_loop(0, chunks_per_tile // 2 - 1, body, 0)
        jl0 = chunks_per_tile - 2
        jl1 = chunks_per_tile - 1
        gwait(jl0, 0)
        sstart(jl0, 0)
        gwait(jl1, 1)
        sstart(jl1, 1)
        swait(jl0, 0)
        swait(jl1, 1)
        plsc.subcore_barrier()
        pltpu.sync_copy(acc_sh.at[pl.ds(base_row, rows_per_tile)],
                        out_hbm.at[cid, pl.ds(base_row, rows_per_tile)])

    return edge_kernel


def _k1_body(degp_ref, x_ref, w1_ref, g1_ref, d_ref):
    deg = degp_ref[0] + degp_ref[1] + 1.0
    d = lax.rsqrt(deg)
    h = jnp.dot(x_ref[:], w1_ref[:], preferred_element_type=jnp.float32)
    g1_ref[:] = h * d
    d_ref[:] = d


def _k2_body(p_ref, g1_ref, d_ref, b1_ref, w2_ref, g2_ref):
    tot = p_ref[0] + p_ref[1] - g1_ref[:]
    a1 = jnp.maximum(d_ref[:] * tot + b1_ref[:], 0.0)
    h2 = jnp.dot(a1, w2_ref[:], preferred_element_type=jnp.float32)
    g2_ref[:] = h2 * d_ref[:]


def _k3_body(n_acc, p_ref, g2_ref, d_ref, b2_ref, batch_ref, wout_ref,
             bout_ref, out_ref):
    tot = p_ref[0] + p_ref[1] - g2_ref[:]
    a2 = jnp.maximum(d_ref[:] * tot + b2_ref[:], 0.0)
    ids = lax.broadcasted_iota(jnp.int32, (_B, n_acc), 0)
    oneh = (ids == batch_ref[:]).astype(jnp.float32)
    sums = jnp.dot(oneh, a2, preferred_element_type=jnp.float32)
    counts = jnp.sum(oneh, axis=1, keepdims=True)
    pooled = sums / jnp.maximum(counts, 1.0)
    out_ref[:] = (jnp.dot(pooled, wout_ref[:],
                          preferred_element_type=jnp.float32) + bout_ref[:])


def kernel(x, edge_index, batch, W1, b1, W2, b2, Wout, bout):
    n, f_in = x.shape
    e = edge_index.shape[1]
    n_acc = _pad_rows(n)
    e_pad = (e + 2 * _NW * _CH - 1) // (2 * _NW * _CH) * (2 * _NW * _CH)
    n_chunks = e_pad // _CH

    f32 = jnp.float32
    x_pad = jnp.zeros((n_acc, f_in), f32).at[:n].set(x)
    pad_idx = jnp.full((e_pad - e,), n, jnp.int32)
    src_chunks = jnp.concatenate([edge_index[0], pad_idx]).reshape(n_chunks, _CH)
    dst_chunks = jnp.concatenate([edge_index[1], pad_idx]).reshape(n_chunks, _CH)
    batch_pad = jnp.full((1, n_acc), _B, jnp.int32).at[0, :n].set(batch)
    zeros_deg = jnp.zeros((n_acc,), f32)

    deg_p = _make_deg_kernel(n_acc, n_chunks)(zeros_deg, dst_chunks)
    deg_p = deg_p.reshape(_NC, n_acc, 1)

    d_out = W1.shape[1]
    g1, d = pl.pallas_call(
        _k1_body,
        out_shape=[jax.ShapeDtypeStruct((n_acc, d_out), f32),
                   jax.ShapeDtypeStruct((n_acc, 1), f32)],
    )(deg_p, x_pad, W1)

    p1 = _make_edge_kernel(n_acc, d_out, n_chunks)(g1, src_chunks, dst_chunks)

    d2_out = W2.shape[1]
    g2 = pl.pallas_call(
        _k2_body,
        out_shape=jax.ShapeDtypeStruct((n_acc, d2_out), f32),
    )(p1, g1, d, b1.reshape(1, -1), W2)

    p2 = _make_edge_kernel(n_acc, d2_out, n_chunks)(g2, src_chunks, dst_chunks)

    out = pl.pallas_call(
        functools.partial(_k3_body, n_acc),
        out_shape=jax.ShapeDtypeStruct((_B, Wout.shape[1]), f32),
    )(p2, g2, d, b2.reshape(1, -1), batch_pad, Wout, bout.reshape(1, -1))
    return out


# speedup vs baseline: 42.8606x; 1.9880x over previous
import functools

import jax
import jax.numpy as jnp
from jax import lax
from jax.experimental import pallas as pl
from jax.experimental.pallas import tpu as pltpu
from jax.experimental.pallas import tpu_sc as plsc

_NC = 2
_NS = 16
_NW = _NC * _NS
_B = 64


def _pad_rows(n):
    return (n + 127) // 128 * 128


@functools.cache
def _make_deg_kernel(n_acc, n_chunks, ch):
    mesh = plsc.VectorSubcoreMesh(core_axis_name="c", subcore_axis_name="s",
                                  num_cores=_NC, num_subcores=_NS)
    chunks_per_tile = n_chunks // _NW
    rows_per_tile = n_acc // _NS

    @functools.partial(
        pl.kernel,
        mesh=mesh,
        out_type=jax.ShapeDtypeStruct((_NC * n_acc,), jnp.float32),
        scratch_types=[
            pltpu.VMEM((chunks_per_tile, ch), jnp.int32),
            pltpu.VMEM((ch,), jnp.float32),
            pltpu.VMEM((n_acc // _NS,), jnp.float32),
            pltpu.VMEM_SHARED((n_acc,), jnp.float32),
            pltpu.SemaphoreType.DMA,
        ],
        compiler_params=pltpu.CompilerParams(use_tc_tiling_on_sc=False),
    )
    def deg_kernel(zeros_hbm, dst_hbm, out_hbm, dst_all, ones_v, tmp_v,
                   acc_sh, sem):
        cid = lax.axis_index("c")
        sid = lax.axis_index("s")
        base_row = sid * rows_per_tile
        wid = sid * _NC + cid
        chunk0 = wid * chunks_per_tile
        pltpu.sync_copy(dst_hbm.at[pl.ds(chunk0, chunks_per_tile)], dst_all)
        pltpu.sync_copy(zeros_hbm.at[pl.ds(base_row, rows_per_tile)], tmp_v)
        pltpu.sync_copy(tmp_v, acc_sh.at[pl.ds(base_row, rows_per_tile)])
        for i in range(ch // 16):
            ones_v[pl.ds(i * 16, 16)] = jnp.ones((16,), jnp.float32)
        plsc.subcore_barrier()

        def fire(j, carry):
            pltpu.async_copy(ones_v, acc_sh.at[dst_all.at[j]], sem, add=True)
            return carry

        lax.fori_loop(0, chunks_per_tile, fire, 0)

        def drain(j, carry):
            pltpu.make_async_copy(ones_v, acc_sh.at[dst_all.at[0]], sem).wait()
            return carry

        lax.fori_loop(0, chunks_per_tile, drain, 0)
        plsc.subcore_barrier()
        pltpu.sync_copy(acc_sh.at[pl.ds(base_row, rows_per_tile)], tmp_v)
        pltpu.sync_copy(tmp_v,
                        out_hbm.at[pl.ds(cid * n_acc + base_row, rows_per_tile)])

    return deg_kernel


@functools.cache
def _make_edge_kernel(n_acc, d_feat, n_chunks, ch):
    mesh = plsc.VectorSubcoreMesh(core_axis_name="c", subcore_axis_name="s",
                                  num_cores=_NC, num_subcores=_NS)
    chunks_per_tile = n_chunks // _NW
    rows_per_tile = n_acc // _NS

    @functools.partial(
        pl.kernel,
        mesh=mesh,
        out_type=jax.ShapeDtypeStruct((_NC, n_acc, d_feat), jnp.float32),
        scratch_types=[
            pltpu.VMEM((chunks_per_tile, ch), jnp.int32),
            pltpu.VMEM((chunks_per_tile, ch), jnp.int32),
            pltpu.VMEM((ch, d_feat), jnp.float32),
            pltpu.VMEM((ch, d_feat), jnp.float32),
            pltpu.VMEM_SHARED((n_acc, d_feat), jnp.float32),
            pltpu.VMEM_SHARED((n_acc, d_feat), jnp.float32),
            pltpu.SemaphoreType.DMA,
            pltpu.SemaphoreType.DMA,
            pltpu.SemaphoreType.DMA,
            pltpu.SemaphoreType.DMA,
        ],
        compiler_params=pltpu.CompilerParams(use_tc_tiling_on_sc=False),
    )
    def edge_kernel(g_hbm, src_hbm, dst_hbm, out_hbm,
                    src_all, dst_all, rows0, rows1,
                    acc_sh, g_sh,
                    gsem0, gsem1, ssem0, ssem1):
        cid = lax.axis_index("c")
        sid = lax.axis_index("s")
        base_row = sid * rows_per_tile
        wid = sid * _NC + cid
        chunk0 = wid * chunks_per_tile
        rows = (rows0, rows1)
        gsem = (gsem0, gsem1)
        ssem = (ssem0, ssem1)
        pltpu.sync_copy(src_hbm.at[pl.ds(chunk0, chunks_per_tile)], src_all)
        pltpu.sync_copy(dst_hbm.at[pl.ds(chunk0, chunks_per_tile)], dst_all)
        row_sl = pl.ds(base_row, rows_per_tile)

        @pl.when(cid == 0)
        def _():
            pltpu.sync_copy(g_hbm.at[row_sl], acc_sh.at[row_sl])

        @pl.when(cid == 1)
        def _():
            zv = jnp.zeros((16,), jnp.float32)

            def zbody(r, c):
                for k in range(d_feat // 16):
                    rows0[r, pl.ds(k * 16, 16)] = zv
                return c

            lax.fori_loop(0, ch, zbody, 0)
            nfull = rows_per_tile // ch
            for q in range(nfull):
                pltpu.sync_copy(rows0,
                                acc_sh.at[pl.ds(base_row + q * ch, ch)])
            rem = rows_per_tile - nfull * ch
            if rem:
                pltpu.sync_copy(rows0.at[pl.ds(0, rem)],
                                acc_sh.at[pl.ds(base_row + nfull * ch, rem)])

        pltpu.sync_copy(g_hbm.at[row_sl], g_sh.at[row_sl])
        plsc.subcore_barrier()

        def gstart(j, b):
            pltpu.async_copy(g_sh.at[src_all.at[j]], rows[b], gsem[b])

        def gwait(j, b):
            pltpu.make_async_copy(g_sh.at[src_all.at[j]], rows[b],
                                  gsem[b]).wait()

        def sstart(j, b):
            pltpu.async_copy(rows[b], acc_sh.at[dst_all.at[j]], ssem[b],
                             add=True)

        def swait(j, b):
            pltpu.make_async_copy(rows[b], acc_sh.at[dst_all.at[j]],
                                  ssem[b]).wait()

        nb = 2
        for b in range(nb):
            gstart(b, b)

        def body(k, carry):
            j = nb * k
            for b in range(nb):
                gwait(j + b, b)
                sstart(j + b, b)
            for b in range(nb):
                swait(j + b, b)
                gstart(j + b + nb, b)
            return carry

        lax.fori_loop(0, chunks_per_tile // nb - 1, body, 0)
        jl = chunks_per_tile - nb
        for b in range(nb):
            gwait(jl + b, b)
            sstart(jl + b, b)
        for b in range(nb):
            swait(jl + b, b)
        plsc.subcore_barrier()
        pltpu.sync_copy(acc_sh.at[pl.ds(base_row, rows_per_tile)],
                        out_hbm.at[cid, pl.ds(base_row, rows_per_tile)])

    return edge_kernel


def _k0_body(x_ref, w1_ref, h_ref):
    h_ref[:] = jnp.dot(x_ref[:], w1_ref[:], preferred_element_type=jnp.float32)


def _k1_body(degp_ref, h_ref, g1_ref, d_ref):
    deg = degp_ref[0] + degp_ref[1] + 1.0
    d = lax.rsqrt(deg)
    g1_ref[:] = h_ref[:] * d
    d_ref[:] = d


def _k2_body(p_ref, d_ref, b1_ref, w2_ref, g2_ref):
    tot = p_ref[0] + p_ref[1]
    a1 = jnp.maximum(d_ref[:] * tot + b1_ref[:], 0.0)
    h2 = jnp.dot(a1, w2_ref[:], preferred_element_type=jnp.float32)
    g2_ref[:] = h2 * d_ref[:]


def _k3_body(n_acc, p_ref, d_ref, b2_ref, batch_ref, wout_ref,
             bout_ref, out_ref):
    tot = p_ref[0] + p_ref[1]
    a2 = jnp.maximum(d_ref[:] * tot + b2_ref[:], 0.0)
    ids = lax.broadcasted_iota(jnp.int32, (_B, n_acc), 0)
    oneh = (ids == batch_ref[:]).astype(jnp.float32)
    sums = jnp.dot(oneh, a2, preferred_element_type=jnp.float32)
    counts = jnp.sum(oneh, axis=1, keepdims=True)
    pooled = sums / jnp.maximum(counts, 1.0)
    out_ref[:] = (jnp.dot(pooled, wout_ref[:],
                          preferred_element_type=jnp.float32) + bout_ref[:])


def kernel(x, edge_index, batch, W1, b1, W2, b2, Wout, bout):
    n, f_in = x.shape
    e = edge_index.shape[1]
    n_acc = _pad_rows(n)
    ch_deg, ch1, ch2 = 512, 128, 256
    grp = max(_NW * ch_deg, 2 * _NW * ch1, 2 * _NW * ch2)
    e_pad = (e + grp - 1) // grp * grp

    f32 = jnp.float32
    x_pad = jnp.zeros((n_acc, f_in), f32).at[:n].set(x)
    pad_idx = jnp.full((e_pad - e,), n, jnp.int32)
    src_flat = jnp.concatenate([edge_index[0], pad_idx])
    dst_flat = jnp.concatenate([edge_index[1], pad_idx])
    batch_pad = jnp.full((1, n_acc), _B, jnp.int32).at[0, :n].set(batch)
    zeros_deg = jnp.zeros((n_acc,), f32)

    d_out = W1.shape[1]
    h1 = pl.pallas_call(
        _k0_body,
        out_shape=jax.ShapeDtypeStruct((n_acc, d_out), f32),
    )(x_pad, W1)

    deg_p = _make_deg_kernel(n_acc, e_pad // ch_deg, ch_deg)(
        zeros_deg, dst_flat.reshape(e_pad // ch_deg, ch_deg))
    deg_p = deg_p.reshape(_NC, n_acc, 1)

    g1, d = pl.pallas_call(
        _k1_body,
        out_shape=[jax.ShapeDtypeStruct((n_acc, d_out), f32),
                   jax.ShapeDtypeStruct((n_acc, 1), f32)],
    )(deg_p, h1)

    p1 = _make_edge_kernel(n_acc, d_out, e_pad // ch1, ch1)(
        g1, src_flat.reshape(e_pad // ch1, ch1),
        dst_flat.reshape(e_pad // ch1, ch1))

    d2_out = W2.shape[1]
    g2 = pl.pallas_call(
        _k2_body,
        out_shape=jax.ShapeDtypeStruct((n_acc, d2_out), f32),
    )(p1, d, b1.reshape(1, -1), W2)

    p2 = _make_edge_kernel(n_acc, d2_out, e_pad // ch2, ch2)(
        g2, src_flat.reshape(e_pad // ch2, ch2),
        dst_flat.reshape(e_pad // ch2, ch2))

    out = pl.pallas_call(
        functools.partial(_k3_body, n_acc),
        out_shape=jax.ShapeDtypeStruct((_B, Wout.shape[1]), f32),
    )(p2, d, b2.reshape(1, -1), batch_pad, Wout, bout.reshape(1, -1))
    return out
